# baseline (speedup 1.0000x reference)
import functools

import jax
import jax.numpy as jnp
from jax import lax
from jax.experimental import pallas as pl
from jax.experimental.pallas import tpu as pltpu
from jax.experimental.pallas import tpu_sc as plsc

INPUT_DIM = 128
CONTEXT_DIM = 8
BATCH = 16384
NUM_CTX = 2 ** CONTEXT_DIM
PAIRS = INPUT_DIM // 2

NUM_CORES = 2
NUM_SUBCORES = 16
LANES = 16
NUM_WORKERS = NUM_CORES * NUM_SUBCORES
CHUNK = BATCH // NUM_WORKERS

TC_BLOCK = 4096
P_BLOCK = 32
N_SPLIT = 4


def _ctx_body(vt_ref, si_ref, b_ref, bc_ref, w_ref, ctx_ref, wpk_ref):
    proj = lax.dot_general(
        vt_ref[...], si_ref[...],
        dimension_numbers=(((0,), (0,)), ((), ())),
        preferred_element_type=jnp.float32,
    )
    bits = (proj > b_ref[...]).astype(jnp.float32)
    ctx_ref[...] = jnp.sum(bits * bc_ref[...], axis=0).astype(jnp.int32)

    @pl.when(pl.program_id(0) == 0)
    def _pack():
        lo = lax.bitcast_convert_type(
            w_ref[0:PAIRS, :].astype(jnp.bfloat16), jnp.uint16)
        hi = lax.bitcast_convert_type(
            w_ref[PAIRS:INPUT_DIM, :].astype(jnp.bfloat16), jnp.uint16)
        pk = lo.astype(jnp.uint32) | (hi.astype(jnp.uint32) << 16)
        wpk_ref[...] = lax.bitcast_convert_type(pk, jnp.int32)


def _context_ids(side_information, vt, b, boolean_converter, weights):
    side_dim = side_information.shape[0]
    return pl.pallas_call(
        _ctx_body,
        grid=(BATCH // TC_BLOCK,),
        in_specs=[
            pl.BlockSpec((side_dim, CONTEXT_DIM), lambda i: (0, 0)),
            pl.BlockSpec((side_dim, TC_BLOCK), lambda i: (0, i)),
            pl.BlockSpec((CONTEXT_DIM, 1), lambda i: (0, 0)),
            pl.BlockSpec((CONTEXT_DIM, 1), lambda i: (0, 0)),
            pl.BlockSpec((INPUT_DIM, NUM_CTX), lambda i: (0, 0)),
        ],
        out_specs=[
            pl.BlockSpec((TC_BLOCK,), lambda i: (i,)),
            pl.BlockSpec((PAIRS, NUM_CTX), lambda i: (0, 0)),
        ],
        out_shape=[
            jax.ShapeDtypeStruct((BATCH,), jnp.int32),
            jax.ShapeDtypeStruct((PAIRS, NUM_CTX), jnp.int32),
        ],
    )(vt, side_information, b, boolean_converter, weights)


def _gln_sc_body(w_hbm, ctx_hbm, lp_hbm, out_hbm,
                 w_ts, ctx_ts, lp_ts, out_ts, sem_w, sem_c, *sem_l):
    wid = lax.axis_index("s") * NUM_CORES + lax.axis_index("c")
    base = wid * CHUNK
    part = CHUNK // N_SPLIT
    cp_w = pltpu.async_copy(w_hbm, w_ts, sem_w)
    cp_c = pltpu.async_copy(ctx_hbm.at[pl.ds(base, CHUNK)], ctx_ts, sem_c)
    cp_l = [
        pltpu.async_copy(
            lp_hbm.at[:, pl.ds(base + q * part, part)],
            lp_ts.at[:, q * part:(q + 1) * part], sem_l[q])
        for q in range(N_SPLIT)
    ]
    mask_hi = jnp.full((LANES,), -65536, jnp.int32)

    def group(j, carry):
        col = j * LANES
        cvec = ctx_ts[pl.ds(col, LANES)]

        def pblock(k, accs):
            a0, a1, a2, a3 = accs
            acc4 = [a0, a1, a2, a3]
            for pp in range(P_BLOCK):
                p = k * P_BLOCK + pp
                g = plsc.load_gather(
                    w_ts, [jnp.full((LANES,), 0, jnp.int32) + p, cvec])
                w_lo = plsc.bitcast(lax.shift_left(g, 16), jnp.float32)
                w_hi = plsc.bitcast(lax.bitwise_and(g, mask_hi), jnp.float32)
                x_lo = lp_ts[p, pl.ds(col, LANES)]
                x_hi = lp_ts[p + PAIRS, pl.ds(col, LANES)]
                acc4[(2 * pp) % 4] = acc4[(2 * pp) % 4] + w_lo * x_lo
                acc4[(2 * pp + 1) % 4] = acc4[(2 * pp + 1) % 4] + w_hi * x_hi
            return tuple(acc4)

        zero = jnp.zeros((LANES,), jnp.float32)
        a0, a1, a2, a3 = lax.fori_loop(
            0, PAIRS // P_BLOCK, pblock, (zero, zero, zero, zero))
        out_ts[pl.ds(col, LANES)] = (a0 + a1) + (a2 + a3)
        return carry

    cp_w.wait()
    cp_c.wait()
    gpp = part // LANES
    for q in range(N_SPLIT):
        cp_l[q].wait()
        lax.fori_loop(q * gpp, (q + 1) * gpp, group, 0)
    pltpu.sync_copy(out_ts, out_hbm.at[pl.ds(base, CHUNK)])


@functools.cache
def _gln_sc():
    return pl.kernel(
        _gln_sc_body,
        out_type=jax.ShapeDtypeStruct((BATCH,), jnp.float32),
        mesh=plsc.VectorSubcoreMesh(
            core_axis_name="c", subcore_axis_name="s",
            num_cores=NUM_CORES, num_subcores=NUM_SUBCORES,
        ),
        scratch_types=[
            pltpu.VMEM((PAIRS, NUM_CTX), jnp.int32),
            pltpu.VMEM((CHUNK,), jnp.int32),
            pltpu.VMEM((INPUT_DIM, CHUNK), jnp.float32),
            pltpu.VMEM((CHUNK,), jnp.float32),
        ] + [pltpu.SemaphoreType.DMA] * (2 + N_SPLIT),
        compiler_params=pltpu.CompilerParams(
            use_tc_tiling_on_sc=True, needs_layout_passes=False,
        ),
    )


def kernel(logit_previous, side_information, v, b, weights, boolean_converter):
    ctx, wpk = _context_ids(side_information, v, b, boolean_converter, weights)
    return _gln_sc()(wpk, ctx, logit_previous)


# speedup vs baseline: 1.0483x; 1.0483x over previous
import functools

import jax
import jax.numpy as jnp
from jax import lax
from jax.experimental import pallas as pl
from jax.experimental.pallas import tpu as pltpu
from jax.experimental.pallas import tpu_sc as plsc

INPUT_DIM = 128
CONTEXT_DIM = 8
BATCH = 16384
NUM_CTX = 2 ** CONTEXT_DIM
PAIRS = INPUT_DIM // 2

NUM_CORES = 2
NUM_SUBCORES = 16
LANES = 16
NUM_WORKERS = NUM_CORES * NUM_SUBCORES
CHUNK = BATCH // NUM_WORKERS

TC_BLOCK = 4096
P_BLOCK = 32
N_SPLIT = 4


def _ctx_body(vt_ref, si_ref, b_ref, bc_ref, w_ref, ctx_ref, wpk_ref):
    proj = lax.dot_general(
        vt_ref[...], si_ref[...],
        dimension_numbers=(((1,), (0,)), ((), ())),
        preferred_element_type=jnp.float32,
    )
    bits = (proj > b_ref[...]).astype(jnp.float32)
    ctx_ref[...] = jnp.sum(bits * bc_ref[...], axis=0).astype(jnp.int32)

    @pl.when(pl.program_id(0) == 0)
    def _pack():
        lo = lax.bitcast_convert_type(
            w_ref[0:PAIRS, :].astype(jnp.bfloat16), jnp.uint16)
        hi = lax.bitcast_convert_type(
            w_ref[PAIRS:INPUT_DIM, :].astype(jnp.bfloat16), jnp.uint16)
        pk = lo.astype(jnp.uint32) | (hi.astype(jnp.uint32) << 16)
        wpk_ref[...] = lax.bitcast_convert_type(pk, jnp.int32)


def _context_ids(side_information, vt, b, boolean_converter, weights):
    side_dim = side_information.shape[0]
    return pl.pallas_call(
        _ctx_body,
        grid=(BATCH // TC_BLOCK,),
        in_specs=[
            pl.BlockSpec((CONTEXT_DIM, side_dim), lambda i: (0, 0)),
            pl.BlockSpec((side_dim, TC_BLOCK), lambda i: (0, i)),
            pl.BlockSpec((CONTEXT_DIM, 1), lambda i: (0, 0)),
            pl.BlockSpec((CONTEXT_DIM, 1), lambda i: (0, 0)),
            pl.BlockSpec((INPUT_DIM, NUM_CTX), lambda i: (0, 0)),
        ],
        out_specs=[
            pl.BlockSpec((TC_BLOCK,), lambda i: (i,)),
            pl.BlockSpec((PAIRS, NUM_CTX), lambda i: (0, 0)),
        ],
        out_shape=[
            jax.ShapeDtypeStruct((BATCH,), jnp.int32),
            jax.ShapeDtypeStruct((PAIRS, NUM_CTX), jnp.int32),
        ],
    )(vt, side_information, b, boolean_converter, weights)


def _gln_sc_body(w_hbm, ctx_hbm, lp_hbm, out_hbm,
                 w_ts, ctx_ts, lp_ts, out_ts, sem_w, sem_c, *sem_l):
    wid = lax.axis_index("s") * NUM_CORES + lax.axis_index("c")
    base = wid * CHUNK
    part = CHUNK // N_SPLIT
    cp_w = pltpu.async_copy(w_hbm, w_ts, sem_w)
    cp_c = pltpu.async_copy(ctx_hbm.at[pl.ds(base, CHUNK)], ctx_ts, sem_c)
    cp_l = [
        pltpu.async_copy(
            lp_hbm.at[:, pl.ds(base + q * part, part)],
            lp_ts.at[:, q * part:(q + 1) * part], sem_l[q])
        for q in range(N_SPLIT)
    ]
    mask_hi = jnp.full((LANES,), -65536, jnp.int32)

    def group(j, carry):
        col = j * LANES
        cvec = ctx_ts[pl.ds(col, LANES)]

        def pblock(k, accs):
            a0, a1, a2, a3 = accs
            acc4 = [a0, a1, a2, a3]
            for pp in range(P_BLOCK):
                p = k * P_BLOCK + pp
                g = plsc.load_gather(
                    w_ts, [jnp.full((LANES,), 0, jnp.int32) + p, cvec])
                w_lo = plsc.bitcast(lax.shift_left(g, 16), jnp.float32)
                w_hi = plsc.bitcast(lax.bitwise_and(g, mask_hi), jnp.float32)
                x_lo = lp_ts[p, pl.ds(col, LANES)]
                x_hi = lp_ts[p + PAIRS, pl.ds(col, LANES)]
                acc4[(2 * pp) % 4] = acc4[(2 * pp) % 4] + w_lo * x_lo
                acc4[(2 * pp + 1) % 4] = acc4[(2 * pp + 1) % 4] + w_hi * x_hi
            return tuple(acc4)

        zero = jnp.zeros((LANES,), jnp.float32)
        a0, a1, a2, a3 = lax.fori_loop(
            0, PAIRS // P_BLOCK, pblock, (zero, zero, zero, zero))
        out_ts[pl.ds(col, LANES)] = (a0 + a1) + (a2 + a3)
        return carry

    cp_w.wait()
    cp_c.wait()
    gpp = part // LANES
    for q in range(N_SPLIT):
        cp_l[q].wait()
        lax.fori_loop(q * gpp, (q + 1) * gpp, group, 0)
    pltpu.sync_copy(out_ts, out_hbm.at[pl.ds(base, CHUNK)])


@functools.cache
def _gln_sc():
    return pl.kernel(
        _gln_sc_body,
        out_type=jax.ShapeDtypeStruct((BATCH,), jnp.float32),
        mesh=plsc.VectorSubcoreMesh(
            core_axis_name="c", subcore_axis_name="s",
            num_cores=NUM_CORES, num_subcores=NUM_SUBCORES,
        ),
        scratch_types=[
            pltpu.VMEM((PAIRS, NUM_CTX), jnp.int32),
            pltpu.VMEM((CHUNK,), jnp.int32),
            pltpu.VMEM((INPUT_DIM, CHUNK), jnp.float32),
            pltpu.VMEM((CHUNK,), jnp.float32),
        ] + [pltpu.SemaphoreType.DMA] * (2 + N_SPLIT),
        compiler_params=pltpu.CompilerParams(
            use_tc_tiling_on_sc=True, needs_layout_passes=False,
        ),
    )


def kernel(logit_previous, side_information, v, b, weights, boolean_converter):
    ctx, wpk = _context_ids(side_information, v.T, b, boolean_converter, weights)
    return _gln_sc()(wpk, ctx, logit_previous)


# speedup vs baseline: 1.0634x; 1.0144x over previous
import functools

import jax
import jax.numpy as jnp
from jax import lax
from jax.experimental import pallas as pl
from jax.experimental.pallas import tpu as pltpu
from jax.experimental.pallas import tpu_sc as plsc

INPUT_DIM = 128
CONTEXT_DIM = 8
BATCH = 16384
NUM_CTX = 2 ** CONTEXT_DIM
PAIRS = INPUT_DIM // 2

NUM_CORES = 2
NUM_SUBCORES = 16
LANES = 16
NUM_WORKERS = NUM_CORES * NUM_SUBCORES
CHUNK = BATCH // NUM_WORKERS

TC_BLOCK = 4096
P_BLOCK = 16
N_SPLIT = 4


def _ctx_body(vt_ref, si_ref, b_ref, bc_ref, w_ref, ctx_ref, wpk_ref):
    proj = lax.dot_general(
        vt_ref[...], si_ref[...],
        dimension_numbers=(((1,), (0,)), ((), ())),
        preferred_element_type=jnp.float32,
    )
    bits = (proj > b_ref[...]).astype(jnp.float32)
    ctx_ref[...] = jnp.sum(bits * bc_ref[...], axis=0).astype(jnp.int32)

    @pl.when(pl.program_id(0) == 0)
    def _pack():
        lo = lax.bitcast_convert_type(
            w_ref[0:PAIRS, :].astype(jnp.bfloat16), jnp.uint16)
        hi = lax.bitcast_convert_type(
            w_ref[PAIRS:INPUT_DIM, :].astype(jnp.bfloat16), jnp.uint16)
        pk = lo.astype(jnp.uint32) | (hi.astype(jnp.uint32) << 16)
        wpk_ref[...] = lax.bitcast_convert_type(pk, jnp.int32)


def _context_ids(side_information, vt, b, boolean_converter, weights):
    side_dim = side_information.shape[0]
    return pl.pallas_call(
        _ctx_body,
        grid=(BATCH // TC_BLOCK,),
        in_specs=[
            pl.BlockSpec((CONTEXT_DIM, side_dim), lambda i: (0, 0)),
            pl.BlockSpec((side_dim, TC_BLOCK), lambda i: (0, i)),
            pl.BlockSpec((CONTEXT_DIM, 1), lambda i: (0, 0)),
            pl.BlockSpec((CONTEXT_DIM, 1), lambda i: (0, 0)),
            pl.BlockSpec((INPUT_DIM, NUM_CTX), lambda i: (0, 0)),
        ],
        out_specs=[
            pl.BlockSpec((TC_BLOCK,), lambda i: (i,)),
            pl.BlockSpec((PAIRS, NUM_CTX), lambda i: (0, 0)),
        ],
        out_shape=[
            jax.ShapeDtypeStruct((BATCH,), jnp.int32),
            jax.ShapeDtypeStruct((PAIRS, NUM_CTX), jnp.int32),
        ],
    )(vt, side_information, b, boolean_converter, weights)


def _gln_sc_body(w_hbm, ctx_hbm, lp_hbm, out_hbm,
                 w_ts, ctx_ts, lp_ts, out_ts, sem_w, sem_c, *sem_l):
    wid = lax.axis_index("s") * NUM_CORES + lax.axis_index("c")
    base = wid * CHUNK
    part = CHUNK // N_SPLIT
    cp_w = pltpu.async_copy(w_hbm, w_ts, sem_w)
    cp_c = pltpu.async_copy(ctx_hbm.at[pl.ds(base, CHUNK)], ctx_ts, sem_c)
    cp_l = [
        pltpu.async_copy(
            lp_hbm.at[:, pl.ds(base + q * part, part)],
            lp_ts.at[:, q * part:(q + 1) * part], sem_l[q])
        for q in range(N_SPLIT)
    ]
    mask_hi = jnp.full((LANES,), -65536, jnp.int32)

    def group(j, carry):
        col = j * LANES
        cvec = ctx_ts[pl.ds(col, LANES)]

        def pblock(k, accs):
            a0, a1, a2, a3 = accs
            acc4 = [a0, a1, a2, a3]
            for pp in range(P_BLOCK):
                p = k * P_BLOCK + pp
                g = plsc.load_gather(
                    w_ts, [jnp.full((LANES,), 0, jnp.int32) + p, cvec])
                w_lo = plsc.bitcast(lax.shift_left(g, 16), jnp.float32)
                w_hi = plsc.bitcast(lax.bitwise_and(g, mask_hi), jnp.float32)
                x_lo = lp_ts[p, pl.ds(col, LANES)]
                x_hi = lp_ts[p + PAIRS, pl.ds(col, LANES)]
                acc4[(2 * pp) % 4] = acc4[(2 * pp) % 4] + w_lo * x_lo
                acc4[(2 * pp + 1) % 4] = acc4[(2 * pp + 1) % 4] + w_hi * x_hi
            return tuple(acc4)

        zero = jnp.zeros((LANES,), jnp.float32)
        a0, a1, a2, a3 = lax.fori_loop(
            0, PAIRS // P_BLOCK, pblock, (zero, zero, zero, zero))
        out_ts[pl.ds(col, LANES)] = (a0 + a1) + (a2 + a3)
        return carry

    cp_w.wait()
    cp_c.wait()
    gpp = part // LANES
    for q in range(N_SPLIT):
        cp_l[q].wait()
        lax.fori_loop(q * gpp, (q + 1) * gpp, group, 0)
    pltpu.sync_copy(out_ts, out_hbm.at[pl.ds(base, CHUNK)])


@functools.cache
def _gln_sc():
    return pl.kernel(
        _gln_sc_body,
        out_type=jax.ShapeDtypeStruct((BATCH,), jnp.float32),
        mesh=plsc.VectorSubcoreMesh(
            core_axis_name="c", subcore_axis_name="s",
            num_cores=NUM_CORES, num_subcores=NUM_SUBCORES,
        ),
        scratch_types=[
            pltpu.VMEM((PAIRS, NUM_CTX), jnp.int32),
            pltpu.VMEM((CHUNK,), jnp.int32),
            pltpu.VMEM((INPUT_DIM, CHUNK), jnp.float32),
            pltpu.VMEM((CHUNK,), jnp.float32),
        ] + [pltpu.SemaphoreType.DMA] * (2 + N_SPLIT),
        compiler_params=pltpu.CompilerParams(
            use_tc_tiling_on_sc=True, needs_layout_passes=False,
        ),
    )


def kernel(logit_previous, side_information, v, b, weights, boolean_converter):
    ctx, wpk = _context_ids(side_information, v.T, b, boolean_converter, weights)
    return _gln_sc()(wpk, ctx, logit_previous)
